# baseline (speedup 1.0000x reference)
import functools

import jax
import jax.numpy as jnp
from jax import lax
from jax.experimental import pallas as pl
from jax.experimental.pallas import tpu as pltpu
from jax.experimental.pallas import tpu_sc as plsc

_D = 768
_NC = 2
_NS = 16
_NW = _NC * _NS
_B = 4 * 8192
_BPW = _B // _NW
_CHUNK = 64
_NBUF = 2
_NCHUNK = _BPW // _CHUNK


def _make_gather():
    mesh = plsc.VectorSubcoreMesh(core_axis_name="c", subcore_axis_name="s")

    @functools.partial(
        pl.kernel,
        mesh=mesh,
        out_type=jax.ShapeDtypeStruct((_B, _D), jnp.float32),
        scratch_types=[
            pltpu.VMEM((_BPW,), jnp.int32),
        ]
        + [pltpu.VMEM((_CHUNK, _D), jnp.float32) for _ in range(_NBUF)]
        + [
            pltpu.SemaphoreType.DMA,
            pltpu.SemaphoreType.DMA,
        ],
    )
    def gather_kernel(table_hbm, idx_hbm, out_hbm, idx_v, *rest):
        bufs = rest[:_NBUF]
        sem_g, sem_w = rest[_NBUF:]
        wid = lax.axis_index("s") * _NC + lax.axis_index("c")
        base = wid * _BPW
        pltpu.sync_copy(idx_hbm.at[pl.ds(base, _BPW)], idx_v)

        gathers = []
        writes = []

        def start_gather(g):
            gathers.append(
                pltpu.async_copy(
                    table_hbm.at[idx_v.at[pl.ds(g * _CHUNK, _CHUNK)]],
                    bufs[g % _NBUF],
                    sem_g,
                )
            )

        start_gather(0)
        start_gather(1)
        for g in range(_NCHUNK):
            gathers[g].wait()
            writes.append(
                pltpu.async_copy(
                    bufs[g % _NBUF],
                    out_hbm.at[pl.ds(base + g * _CHUNK, _CHUNK)],
                    sem_w,
                )
            )
            if g + 2 < _NCHUNK:
                if g + 2 - _NBUF >= 0:
                    writes[g + 2 - _NBUF].wait()
                start_gather(g + 2)
        for w in writes[max(0, _NCHUNK - _NBUF):]:
            w.wait()

    return gather_kernel


_gather = _make_gather()


def kernel(input_ids, position_ids, pos_table):
    del input_ids
    flat_ids = position_ids.reshape(-1)
    out = _gather(pos_table, flat_ids)
    return out.reshape(position_ids.shape + (pos_table.shape[1],))
